# Initial kernel scaffold; baseline (speedup 1.0000x reference)
import jax
import jax.numpy as jnp
from jax.experimental import pallas as pl


def kernel(x, edge_index, batch, embed_table, W, b):
    raise NotImplementedError("write your pallas kernel here")


# speedup vs baseline: 86.5682x; 86.5682x over previous
import functools

import jax
import jax.numpy as jnp
from jax import lax
from jax.experimental import pallas as pl
from jax.experimental.pallas import tpu as pltpu
from jax.experimental.pallas import tpu_sc as plsc

N_NODES = 10000
N_EDGES = 320000
HIDDEN = 128
NUM_GRAPHS = 64
VOCAB = 28

NP = 10240
NPT = NP // 16
EPT = N_EDGES // 16


def _vrsqrt(d):
    i = lax.bitcast_convert_type(d, jnp.int32)
    i = jnp.int32(0x5F3759DF) - lax.shift_right_arithmetic(i, 1)
    y = lax.bitcast_convert_type(i, jnp.float32)
    for _ in range(3):
        y = y * (1.5 - 0.5 * d * y * y)
    return y


def _sc_body(x_hbm, src_hbm, dst_hbm, bt_hbm, emb_hbm, w_hbm, b_hbm,
             out_hbm,
             embed_v, w_v, b_v, zvec_v, s_v, hist_v, acc_v, src_v, dst_v,
             xv_v, btv_v, mat_v, dinv_v, z_v, out64_v, outm_v,
             sh_part, sh_s, sh_out):
    cid = lax.axis_index("c")
    sid = lax.axis_index("s")
    viota = lax.iota(jnp.int32, 16)
    zf = jnp.zeros((16,), jnp.float32)
    ones = jnp.ones((16,), jnp.float32)
    nbase = sid * NPT
    ebase = sid * EPT

    pltpu.sync_copy(emb_hbm, embed_v.at[pl.ds(0, VOCAB * HIDDEN)])
    pltpu.sync_copy(w_hbm, w_v)
    pltpu.sync_copy(b_hbm, b_v)
    pltpu.sync_copy(src_hbm.at[pl.ds(ebase, EPT)], src_v)
    pltpu.sync_copy(dst_hbm.at[pl.ds(ebase, EPT)], dst_v)
    pltpu.sync_copy(x_hbm.at[pl.ds(nbase, NPT)], xv_v)
    pltpu.sync_copy(bt_hbm.at[pl.ds(nbase, NPT)], btv_v)

    idx0 = viota * HIDDEN
    idx1 = idx0 + 16 * HIDDEN

    def zstep(kk, carry):
        z0, z1 = carry
        wv = w_v[pl.ds(kk * 16, 16)]
        base = kk * 16
        for j in range(16):
            wk = wv[j]
            c0 = plsc.load_gather(embed_v, [idx0 + (base + j)])
            c1 = plsc.load_gather(embed_v, [idx1 + (base + j)])
            z0 = z0 + c0 * wk
            z1 = z1 + c1 * wk
        return (z0, z1)

    zv0, zv1 = lax.fori_loop(0, HIDDEN // 16, zstep, (zf, zf))
    zvec_v[pl.ds(0, 16)] = zv0
    zvec_v[pl.ds(16, 16)] = zv1

    def zero_step(i, c):
        hist_v[pl.ds(i * 16, 16)] = zf
        acc_v[pl.ds(i * 16, 16)] = zf
        return c

    lax.fori_loop(0, NP // 16, zero_step, 0)

    def hstep(i, c):
        dv = dst_v[pl.ds(i * 16, 16)]
        plsc.addupdate_scatter(hist_v, [dv], ones)
        return c

    lax.fori_loop(0, EPT // 16, hstep, 0)

    pltpu.sync_copy(hist_v, sh_part.at[sid])
    plsc.subcore_barrier()
    for t in range(16):
        pltpu.sync_copy(sh_part.at[t, pl.ds(nbase, NPT)], mat_v.at[t])

    def dstep(i, c):
        sl = pl.ds(i * 16, 16)
        a = mat_v[0, sl]
        for t in range(1, 16):
            a = a + mat_v[t, sl]
        d = a + 1.0
        di = _vrsqrt(d)
        dinv_v[sl] = di
        zl = plsc.load_gather(zvec_v, [xv_v[sl]])
        z_v[sl] = zl
        s_v[pl.ds(nbase + i * 16, 16)] = di * zl
        return c

    lax.fori_loop(0, NPT // 16, dstep, 0)
    pltpu.sync_copy(s_v.at[pl.ds(nbase, NPT)], sh_s.at[pl.ds(nbase, NPT)])
    plsc.subcore_barrier()
    pltpu.sync_copy(sh_s, s_v)

    def estep(i, c):
        sl = pl.ds(i * 16, 16)
        m = plsc.load_gather(s_v, [src_v[sl]])
        plsc.addupdate_scatter(acc_v, [dst_v[sl]], m)
        return c

    lax.fori_loop(0, EPT // 16, estep, 0)

    pltpu.sync_copy(acc_v, sh_part.at[sid])
    plsc.subcore_barrier()
    for t in range(16):
        pltpu.sync_copy(sh_part.at[t, pl.ds(nbase, NPT)], mat_v.at[t])
    for q in range(NUM_GRAPHS // 16):
        out64_v[pl.ds(q * 16, 16)] = zf
    bvec = b_v[pl.ds(0, 16)]

    def fstep(i, c):
        sl = pl.ds(i * 16, 16)
        a = mat_v[0, sl]
        for t in range(1, 16):
            a = a + mat_v[t, sl]
        di = dinv_v[sl]
        yv = di * a + di * di * z_v[sl] + bvec
        gid = nbase + i * 16 + viota
        plsc.addupdate_scatter(out64_v, [btv_v[sl]], yv, mask=gid < N_NODES)
        return c

    lax.fori_loop(0, NPT // 16, fstep, 0)

    pltpu.sync_copy(out64_v, sh_out.at[sid, pl.ds(0, NUM_GRAPHS)])
    plsc.subcore_barrier()

    @pl.when(jnp.logical_and(cid == 0, sid == 0))
    def _():
        for t in range(16):
            pltpu.sync_copy(sh_out.at[t, pl.ds(0, NUM_GRAPHS)], outm_v.at[t])
        for q in range(NUM_GRAPHS // 16):
            a = outm_v[0, pl.ds(q * 16, 16)]
            for t in range(1, 16):
                a = a + outm_v[t, pl.ds(q * 16, 16)]
            out64_v[pl.ds(q * 16, 16)] = a
        pltpu.sync_copy(out64_v, out_hbm)


@functools.partial(
    pl.kernel,
    out_type=jax.ShapeDtypeStruct((NUM_GRAPHS,), jnp.float32),
    mesh=plsc.VectorSubcoreMesh(core_axis_name="c", subcore_axis_name="s",
                                num_cores=2, num_subcores=16),
    compiler_params=pltpu.CompilerParams(needs_layout_passes=False),
    scratch_types=[
        pltpu.VMEM((4096,), jnp.float32),
        pltpu.VMEM((HIDDEN,), jnp.float32),
        pltpu.VMEM((16,), jnp.float32),
        pltpu.VMEM((32,), jnp.float32),
        pltpu.VMEM((NP,), jnp.float32),
        pltpu.VMEM((NP,), jnp.float32),
        pltpu.VMEM((NP,), jnp.float32),
        pltpu.VMEM((EPT,), jnp.int32),
        pltpu.VMEM((EPT,), jnp.int32),
        pltpu.VMEM((NPT,), jnp.int32),
        pltpu.VMEM((NPT,), jnp.int32),
        pltpu.VMEM((16, NPT), jnp.float32),
        pltpu.VMEM((NPT,), jnp.float32),
        pltpu.VMEM((NPT,), jnp.float32),
        pltpu.VMEM((NUM_GRAPHS,), jnp.float32),
        pltpu.VMEM((16, NUM_GRAPHS), jnp.float32),
        pltpu.VMEM_SHARED((16, NP), jnp.float32),
        pltpu.VMEM_SHARED((NP,), jnp.float32),
        pltpu.VMEM_SHARED((16, 128), jnp.float32),
    ],
)
def _sc_kernel(x_p, src, dst, batch_p, embed_flat, w_flat, b_bcast, out,
               *scratch):
    _sc_body(x_p, src, dst, batch_p, embed_flat, w_flat, b_bcast, out,
             *scratch)


def kernel(x, edge_index, batch, embed_table, W, b):
    x_p = jnp.pad(x[:, 0], (0, NP - N_NODES))
    batch_p = jnp.pad(batch, (0, NP - N_NODES))
    return _sc_kernel(x_p, edge_index[0], edge_index[1], batch_p,
                      embed_table.reshape(-1), W.reshape(-1),
                      jnp.broadcast_to(b, (16,)))


# speedup vs baseline: 111.0379x; 1.2827x over previous
import functools

import jax
import jax.numpy as jnp
from jax import lax
from jax.experimental import pallas as pl
from jax.experimental.pallas import tpu as pltpu
from jax.experimental.pallas import tpu_sc as plsc

N_NODES = 10000
N_EDGES = 320000
HIDDEN = 128
NUM_GRAPHS = 64
VOCAB = 28

NP = 10240
NPT = NP // 16
EPT = N_EDGES // 16


def _vrsqrt(d):
    i = lax.bitcast_convert_type(d, jnp.int32)
    i = jnp.int32(0x5F3759DF) - lax.shift_right_arithmetic(i, 1)
    y = lax.bitcast_convert_type(i, jnp.float32)
    for _ in range(3):
        y = y * (1.5 - 0.5 * d * y * y)
    return y


def _sc_body(x_hbm, src_hbm, dst_hbm, bt_hbm, emb_hbm, w_hbm, b_hbm,
             out_hbm,
             embed_v, w_v, b_v, zvec_v, s_v, hist_v, acc_v, src_v, dst_v,
             xv_v, btv_v, mat_v, dinv_v, z_v, out64_v, outm_v,
             sh_part, sh_s, sh_out):
    cid = lax.axis_index("c")
    sid = lax.axis_index("s")
    viota = lax.iota(jnp.int32, 16)
    zf = jnp.zeros((16,), jnp.float32)
    ones = jnp.ones((16,), jnp.float32)
    nbase = sid * NPT
    ebase = sid * EPT

    pltpu.sync_copy(emb_hbm, embed_v.at[pl.ds(0, VOCAB * HIDDEN)])
    pltpu.sync_copy(w_hbm, w_v)
    pltpu.sync_copy(b_hbm, b_v)
    pltpu.sync_copy(src_hbm.at[pl.ds(ebase, EPT)], src_v)
    pltpu.sync_copy(dst_hbm.at[pl.ds(ebase, EPT)], dst_v)
    pltpu.sync_copy(x_hbm.at[pl.ds(nbase, NPT)], xv_v)
    pltpu.sync_copy(bt_hbm.at[pl.ds(nbase, NPT)], btv_v)

    idx0 = viota * HIDDEN
    idx1 = idx0 + 16 * HIDDEN

    def zstep(kk, carry):
        z0, z1 = carry
        wv = w_v[pl.ds(kk * 16, 16)]
        base = kk * 16
        for j in range(16):
            wk = wv[j]
            c0 = plsc.load_gather(embed_v, [idx0 + (base + j)])
            c1 = plsc.load_gather(embed_v, [idx1 + (base + j)])
            z0 = z0 + c0 * wk
            z1 = z1 + c1 * wk
        return (z0, z1)

    zv0, zv1 = lax.fori_loop(0, HIDDEN // 16, zstep, (zf, zf))
    zvec_v[pl.ds(0, 16)] = zv0
    zvec_v[pl.ds(16, 16)] = zv1

    @plsc.parallel_loop(0, NP // 16, unroll=8)
    def _(i):
        hist_v[pl.ds(i * 16, 16)] = zf
        acc_v[pl.ds(i * 16, 16)] = zf

    @plsc.parallel_loop(0, EPT // 16, unroll=8)
    def _(i):
        dv = dst_v[pl.ds(i * 16, 16)]
        plsc.addupdate_scatter(hist_v, [dv], ones)

    pltpu.sync_copy(hist_v, sh_part.at[sid])
    plsc.subcore_barrier()
    for t in range(16):
        pltpu.sync_copy(sh_part.at[t, pl.ds(nbase, NPT)], mat_v.at[t])

    @plsc.parallel_loop(0, NPT // 16, unroll=2)
    def _(i):
        sl = pl.ds(i * 16, 16)
        a = mat_v[0, sl]
        for t in range(1, 16):
            a = a + mat_v[t, sl]
        d = a + 1.0
        di = _vrsqrt(d)
        dinv_v[sl] = di
        zl = plsc.load_gather(zvec_v, [xv_v[sl]])
        z_v[sl] = zl
        s_v[pl.ds(nbase + i * 16, 16)] = di * zl
    pltpu.sync_copy(s_v.at[pl.ds(nbase, NPT)], sh_s.at[pl.ds(nbase, NPT)])
    plsc.subcore_barrier()
    pltpu.sync_copy(sh_s, s_v)

    @plsc.parallel_loop(0, EPT // 16, unroll=8)
    def _(i):
        sl = pl.ds(i * 16, 16)
        m = plsc.load_gather(s_v, [src_v[sl]])
        plsc.addupdate_scatter(acc_v, [dst_v[sl]], m)

    pltpu.sync_copy(acc_v, sh_part.at[sid])
    plsc.subcore_barrier()
    for t in range(16):
        pltpu.sync_copy(sh_part.at[t, pl.ds(nbase, NPT)], mat_v.at[t])
    for q in range(NUM_GRAPHS // 16):
        out64_v[pl.ds(q * 16, 16)] = zf
    bvec = b_v[pl.ds(0, 16)]

    @plsc.parallel_loop(0, NPT // 16, unroll=2)
    def _(i):
        sl = pl.ds(i * 16, 16)
        a = mat_v[0, sl]
        for t in range(1, 16):
            a = a + mat_v[t, sl]
        di = dinv_v[sl]
        yv = di * a + di * di * z_v[sl] + bvec
        gid = nbase + i * 16 + viota
        plsc.addupdate_scatter(out64_v, [btv_v[sl]], yv, mask=gid < N_NODES)

    pltpu.sync_copy(out64_v, sh_out.at[sid, pl.ds(0, NUM_GRAPHS)])
    plsc.subcore_barrier()

    @pl.when(jnp.logical_and(cid == 0, sid == 0))
    def _():
        for t in range(16):
            pltpu.sync_copy(sh_out.at[t, pl.ds(0, NUM_GRAPHS)], outm_v.at[t])
        for q in range(NUM_GRAPHS // 16):
            a = outm_v[0, pl.ds(q * 16, 16)]
            for t in range(1, 16):
                a = a + outm_v[t, pl.ds(q * 16, 16)]
            out64_v[pl.ds(q * 16, 16)] = a
        pltpu.sync_copy(out64_v, out_hbm)


@functools.partial(
    pl.kernel,
    out_type=jax.ShapeDtypeStruct((NUM_GRAPHS,), jnp.float32),
    mesh=plsc.VectorSubcoreMesh(core_axis_name="c", subcore_axis_name="s",
                                num_cores=2, num_subcores=16),
    compiler_params=pltpu.CompilerParams(needs_layout_passes=False),
    scratch_types=[
        pltpu.VMEM((4096,), jnp.float32),
        pltpu.VMEM((HIDDEN,), jnp.float32),
        pltpu.VMEM((16,), jnp.float32),
        pltpu.VMEM((32,), jnp.float32),
        pltpu.VMEM((NP,), jnp.float32),
        pltpu.VMEM((NP,), jnp.float32),
        pltpu.VMEM((NP,), jnp.float32),
        pltpu.VMEM((EPT,), jnp.int32),
        pltpu.VMEM((EPT,), jnp.int32),
        pltpu.VMEM((NPT,), jnp.int32),
        pltpu.VMEM((NPT,), jnp.int32),
        pltpu.VMEM((16, NPT), jnp.float32),
        pltpu.VMEM((NPT,), jnp.float32),
        pltpu.VMEM((NPT,), jnp.float32),
        pltpu.VMEM((NUM_GRAPHS,), jnp.float32),
        pltpu.VMEM((16, NUM_GRAPHS), jnp.float32),
        pltpu.VMEM_SHARED((16, NP), jnp.float32),
        pltpu.VMEM_SHARED((NP,), jnp.float32),
        pltpu.VMEM_SHARED((16, 128), jnp.float32),
    ],
)
def _sc_kernel(x_p, src, dst, batch_p, embed_flat, w_flat, b_bcast, out,
               *scratch):
    _sc_body(x_p, src, dst, batch_p, embed_flat, w_flat, b_bcast, out,
             *scratch)


def kernel(x, edge_index, batch, embed_table, W, b):
    x_p = jnp.pad(x[:, 0], (0, NP - N_NODES))
    batch_p = jnp.pad(batch, (0, NP - N_NODES))
    return _sc_kernel(x_p, edge_index[0], edge_index[1], batch_p,
                      embed_table.reshape(-1), W.reshape(-1),
                      jnp.broadcast_to(b, (16,)))


# speedup vs baseline: 116.9557x; 1.0533x over previous
import functools

import jax
import jax.numpy as jnp
from jax import lax
from jax.experimental import pallas as pl
from jax.experimental.pallas import tpu as pltpu
from jax.experimental.pallas import tpu_sc as plsc

N_NODES = 10000
N_EDGES = 320000
HIDDEN = 128
NUM_GRAPHS = 64
VOCAB = 28

NP = 10240
NPT = NP // 16
EPT = N_EDGES // 16


def _vrsqrt(d):
    i = lax.bitcast_convert_type(d, jnp.int32)
    i = jnp.int32(0x5F3759DF) - lax.shift_right_arithmetic(i, 1)
    y = lax.bitcast_convert_type(i, jnp.float32)
    for _ in range(3):
        y = y * (1.5 - 0.5 * d * y * y)
    return y


def _sc_body(x_hbm, src_hbm, dst_hbm, bt_hbm, emb_hbm, w_hbm, b_hbm,
             out_hbm,
             embed_v, w_v, b_v, zvec_v, s_v, hist_v, acc_v, src_v, dst_v,
             xv_v, btv_v, mat_v, dinv_v, z_v, out64_v, outm_v,
             sh_part, sh_s, sh_out):
    cid = lax.axis_index("c")
    sid = lax.axis_index("s")
    viota = lax.iota(jnp.int32, 16)
    zf = jnp.zeros((16,), jnp.float32)
    ones = jnp.ones((16,), jnp.float32)
    nbase = sid * NPT
    ebase = sid * EPT

    pltpu.sync_copy(emb_hbm, embed_v.at[pl.ds(0, VOCAB * HIDDEN)])
    pltpu.sync_copy(w_hbm, w_v)
    pltpu.sync_copy(b_hbm, b_v)
    pltpu.sync_copy(src_hbm.at[pl.ds(ebase, EPT)], src_v)
    pltpu.sync_copy(dst_hbm.at[pl.ds(ebase, EPT)], dst_v)
    pltpu.sync_copy(x_hbm.at[pl.ds(nbase, NPT)], xv_v)
    pltpu.sync_copy(bt_hbm.at[pl.ds(nbase, NPT)], btv_v)

    idx0 = viota * HIDDEN
    idx1 = idx0 + 16 * HIDDEN

    def zstep(kk, carry):
        z0, z1 = carry
        wv = w_v[pl.ds(kk * 16, 16)]
        base = kk * 16
        for j in range(16):
            wk = wv[j]
            c0 = plsc.load_gather(embed_v, [idx0 + (base + j)])
            c1 = plsc.load_gather(embed_v, [idx1 + (base + j)])
            z0 = z0 + c0 * wk
            z1 = z1 + c1 * wk
        return (z0, z1)

    zv0, zv1 = lax.fori_loop(0, HIDDEN // 16, zstep, (zf, zf))
    zvec_v[pl.ds(0, 16)] = zv0
    zvec_v[pl.ds(16, 16)] = zv1

    @plsc.parallel_loop(0, NP // 16, unroll=8)
    def _(i):
        hist_v[pl.ds(i * 16, 16)] = zf
        acc_v[pl.ds(i * 16, 16)] = zf

    @plsc.parallel_loop(0, EPT // 16, unroll=8)
    def _(i):
        dv = dst_v[pl.ds(i * 16, 16)]
        plsc.addupdate_scatter(hist_v, [dv], ones)

    pltpu.sync_copy(hist_v, sh_part.at[sid])
    plsc.subcore_barrier()
    for t in range(16):
        pltpu.sync_copy(sh_part.at[t, pl.ds(nbase, NPT)], mat_v.at[t])

    @plsc.parallel_loop(0, NPT // 16, unroll=2)
    def _(i):
        sl = pl.ds(i * 16, 16)
        a = mat_v[0, sl]
        for t in range(1, 16):
            a = a + mat_v[t, sl]
        d = a + 1.0
        di = _vrsqrt(d)
        dinv_v[sl] = di
        zl = plsc.load_gather(zvec_v, [xv_v[sl]])
        z_v[sl] = zl
        s_v[pl.ds(nbase + i * 16, 16)] = di * zl
    pltpu.sync_copy(s_v.at[pl.ds(nbase, NPT)], sh_s.at[pl.ds(nbase, NPT)])
    plsc.subcore_barrier()
    pltpu.sync_copy(sh_s, s_v)

    @plsc.parallel_loop(0, EPT // 16, unroll=8)
    def _(i):
        sl = pl.ds(i * 16, 16)
        m = plsc.load_gather(s_v, [src_v[sl]])
        plsc.addupdate_scatter(acc_v, [dst_v[sl]], m)

    pltpu.sync_copy(acc_v, sh_part.at[sid])
    plsc.subcore_barrier()
    for t in range(16):
        pltpu.sync_copy(sh_part.at[t, pl.ds(nbase, NPT)], mat_v.at[t])
    for q in range(NUM_GRAPHS // 16):
        out64_v[pl.ds(q * 16, 16)] = zf
    bvec = b_v[pl.ds(0, 16)]

    @plsc.parallel_loop(0, NPT // 16, unroll=2)
    def _(i):
        sl = pl.ds(i * 16, 16)
        a = mat_v[0, sl]
        for t in range(1, 16):
            a = a + mat_v[t, sl]
        di = dinv_v[sl]
        yv = di * a + di * di * z_v[sl] + bvec
        gid = nbase + i * 16 + viota
        plsc.addupdate_scatter(out64_v, [btv_v[sl]], yv, mask=gid < N_NODES)

    pltpu.sync_copy(out64_v, sh_out.at[sid, pl.ds(0, NUM_GRAPHS)])
    plsc.subcore_barrier()

    @pl.when(jnp.logical_and(cid == 0, sid == 0))
    def _():
        for t in range(16):
            pltpu.sync_copy(sh_out.at[t, pl.ds(0, NUM_GRAPHS)], outm_v.at[t])
        for q in range(NUM_GRAPHS // 16):
            a = outm_v[0, pl.ds(q * 16, 16)]
            for t in range(1, 16):
                a = a + outm_v[t, pl.ds(q * 16, 16)]
            out64_v[pl.ds(q * 16, 16)] = a
        pltpu.sync_copy(out64_v, out_hbm)


@functools.partial(
    pl.kernel,
    out_type=jax.ShapeDtypeStruct((NUM_GRAPHS,), jnp.float32),
    mesh=plsc.VectorSubcoreMesh(core_axis_name="c", subcore_axis_name="s",
                                num_cores=1, num_subcores=16),
    compiler_params=pltpu.CompilerParams(needs_layout_passes=False),
    scratch_types=[
        pltpu.VMEM((4096,), jnp.float32),
        pltpu.VMEM((HIDDEN,), jnp.float32),
        pltpu.VMEM((16,), jnp.float32),
        pltpu.VMEM((32,), jnp.float32),
        pltpu.VMEM((NP,), jnp.float32),
        pltpu.VMEM((NP,), jnp.float32),
        pltpu.VMEM((NP,), jnp.float32),
        pltpu.VMEM((EPT,), jnp.int32),
        pltpu.VMEM((EPT,), jnp.int32),
        pltpu.VMEM((NPT,), jnp.int32),
        pltpu.VMEM((NPT,), jnp.int32),
        pltpu.VMEM((16, NPT), jnp.float32),
        pltpu.VMEM((NPT,), jnp.float32),
        pltpu.VMEM((NPT,), jnp.float32),
        pltpu.VMEM((NUM_GRAPHS,), jnp.float32),
        pltpu.VMEM((16, NUM_GRAPHS), jnp.float32),
        pltpu.VMEM_SHARED((16, NP), jnp.float32),
        pltpu.VMEM_SHARED((NP,), jnp.float32),
        pltpu.VMEM_SHARED((16, 128), jnp.float32),
    ],
)
def _sc_kernel(x_p, src, dst, batch_p, embed_flat, w_flat, b_bcast, out,
               *scratch):
    _sc_body(x_p, src, dst, batch_p, embed_flat, w_flat, b_bcast, out,
             *scratch)


def kernel(x, edge_index, batch, embed_table, W, b):
    x_p = jnp.pad(x[:, 0], (0, NP - N_NODES))
    batch_p = jnp.pad(batch, (0, NP - N_NODES))
    return _sc_kernel(x_p, edge_index[0], edge_index[1], batch_p,
                      embed_table.reshape(-1), W.reshape(-1),
                      jnp.broadcast_to(b, (16,)))


# speedup vs baseline: 128.6564x; 1.1000x over previous
import functools

import jax
import jax.numpy as jnp
from jax import lax
from jax.experimental import pallas as pl
from jax.experimental.pallas import tpu as pltpu
from jax.experimental.pallas import tpu_sc as plsc

N_NODES = 10000
N_EDGES = 320000
HIDDEN = 128
NUM_GRAPHS = 64
VOCAB = 28

NP = 10240
NPT = NP // 16
EPT = N_EDGES // 16


def _vrsqrt(d):
    i = lax.bitcast_convert_type(d, jnp.int32)
    i = jnp.int32(0x5F3759DF) - lax.shift_right_arithmetic(i, 1)
    y = lax.bitcast_convert_type(i, jnp.float32)
    for _ in range(3):
        y = y * (1.5 - 0.5 * d * y * y)
    return y


def _sc_body(x_hbm, src_hbm, dst_hbm, bt_hbm, emb_hbm, w_hbm, b_hbm,
             out_hbm,
             embed_v, w_v, b_v, zvec_v, s_v, hist_v, acc_v, src_v, dst_v,
             xv_v, btv_v, mat_v, dinv_v, z_v, out64_v, outm_v,
             sem_src, sem_dst,
             sh_part, sh_s, sh_out):
    sid = lax.axis_index("s")
    viota = lax.iota(jnp.int32, 16)
    zf = jnp.zeros((16,), jnp.float32)
    ones = jnp.ones((16,), jnp.float32)
    nbase = sid * NPT
    ebase = sid * EPT

    with jax.named_scope("ph_stage"):
        cp_dst = pltpu.async_copy(dst_hbm.at[pl.ds(ebase, EPT)], dst_v,
                                  sem_dst)
        cp_src = pltpu.async_copy(src_hbm.at[pl.ds(ebase, EPT)], src_v,
                                  sem_src)
        pltpu.sync_copy(emb_hbm, embed_v.at[pl.ds(0, VOCAB * HIDDEN)])
        pltpu.sync_copy(w_hbm, w_v)
        pltpu.sync_copy(b_hbm, b_v)
        pltpu.sync_copy(x_hbm.at[pl.ds(nbase, NPT)], xv_v)
        pltpu.sync_copy(bt_hbm.at[pl.ds(nbase, NPT)], btv_v)

    with jax.named_scope("ph_zvec"):
        idx0 = viota * HIDDEN
        idx1 = idx0 + 16 * HIDDEN

        def zstep(kk, carry):
            z0, z1 = carry
            wv = w_v[pl.ds(kk * 16, 16)]
            base = kk * 16
            for j in range(16):
                wk = wv[j]
                c0 = plsc.load_gather(embed_v, [idx0 + (base + j)])
                c1 = plsc.load_gather(embed_v, [idx1 + (base + j)])
                z0 = z0 + c0 * wk
                z1 = z1 + c1 * wk
            return (z0, z1)

        zv0, zv1 = lax.fori_loop(0, HIDDEN // 16, zstep, (zf, zf))
        zvec_v[pl.ds(0, 16)] = zv0
        zvec_v[pl.ds(16, 16)] = zv1

    with jax.named_scope("ph_zero"):
        @plsc.parallel_loop(0, NP // 16, unroll=8)
        def _(i):
            hist_v[pl.ds(i * 16, 16)] = zf
            acc_v[pl.ds(i * 16, 16)] = zf

    with jax.named_scope("ph_hist"):
        cp_dst.wait()

        @plsc.parallel_loop(0, EPT // 16, unroll=8)
        def _(i):
            dv = dst_v[pl.ds(i * 16, 16)]
            plsc.addupdate_scatter(hist_v, [dv], ones)

    with jax.named_scope("ph_degcomb"):
        pltpu.sync_copy(hist_v, sh_part.at[sid])
        plsc.subcore_barrier()
        pltpu.sync_copy(sh_part.at[:, pl.ds(nbase, NPT)], mat_v)

        @plsc.parallel_loop(0, NPT // 16, unroll=2)
        def _(i):
            sl = pl.ds(i * 16, 16)
            a = mat_v[0, sl]
            for t in range(1, 16):
                a = a + mat_v[t, sl]
            d = a + 1.0
            di = _vrsqrt(d)
            dinv_v[sl] = di
            zl = plsc.load_gather(zvec_v, [xv_v[sl]])
            z_v[sl] = zl
            s_v[pl.ds(nbase + i * 16, 16)] = di * zl

        pltpu.sync_copy(s_v.at[pl.ds(nbase, NPT)],
                        sh_s.at[pl.ds(nbase, NPT)])
        plsc.subcore_barrier()
        pltpu.sync_copy(sh_s, s_v)

    with jax.named_scope("ph_edge"):
        cp_src.wait()

        @plsc.parallel_loop(0, EPT // 16, unroll=8)
        def _(i):
            sl = pl.ds(i * 16, 16)
            m = plsc.load_gather(s_v, [src_v[sl]])
            plsc.addupdate_scatter(acc_v, [dst_v[sl]], m)

    with jax.named_scope("ph_final"):
        pltpu.sync_copy(acc_v, sh_part.at[sid])
        plsc.subcore_barrier()
        pltpu.sync_copy(sh_part.at[:, pl.ds(nbase, NPT)], mat_v)
        for q in range(NUM_GRAPHS // 16):
            out64_v[pl.ds(q * 16, 16)] = zf
        bvec = b_v[pl.ds(0, 16)]

        @plsc.parallel_loop(0, NPT // 16, unroll=2)
        def _(i):
            sl = pl.ds(i * 16, 16)
            a = mat_v[0, sl]
            for t in range(1, 16):
                a = a + mat_v[t, sl]
            di = dinv_v[sl]
            yv = di * a + di * di * z_v[sl] + bvec
            gid = nbase + i * 16 + viota
            plsc.addupdate_scatter(out64_v, [btv_v[sl]], yv,
                                   mask=gid < N_NODES)

    with jax.named_scope("ph_out"):
        pltpu.sync_copy(out64_v, sh_out.at[sid, pl.ds(0, NUM_GRAPHS)])
        plsc.subcore_barrier()

        @pl.when(sid == 0)
        def _():
            for t in range(16):
                pltpu.sync_copy(sh_out.at[t, pl.ds(0, NUM_GRAPHS)],
                                outm_v.at[t])
            for q in range(NUM_GRAPHS // 16):
                a = outm_v[0, pl.ds(q * 16, 16)]
                for t in range(1, 16):
                    a = a + outm_v[t, pl.ds(q * 16, 16)]
                out64_v[pl.ds(q * 16, 16)] = a
            pltpu.sync_copy(out64_v, out_hbm)


@functools.partial(
    pl.kernel,
    out_type=jax.ShapeDtypeStruct((NUM_GRAPHS,), jnp.float32),
    mesh=plsc.VectorSubcoreMesh(core_axis_name="c", subcore_axis_name="s",
                                num_cores=1, num_subcores=16),
    compiler_params=pltpu.CompilerParams(needs_layout_passes=False),
    scratch_types=[
        pltpu.VMEM((4096,), jnp.float32),
        pltpu.VMEM((HIDDEN,), jnp.float32),
        pltpu.VMEM((16,), jnp.float32),
        pltpu.VMEM((32,), jnp.float32),
        pltpu.VMEM((NP,), jnp.float32),
        pltpu.VMEM((NP,), jnp.float32),
        pltpu.VMEM((NP,), jnp.float32),
        pltpu.VMEM((EPT,), jnp.int32),
        pltpu.VMEM((EPT,), jnp.int32),
        pltpu.VMEM((NPT,), jnp.int32),
        pltpu.VMEM((NPT,), jnp.int32),
        pltpu.VMEM((16, NPT), jnp.float32),
        pltpu.VMEM((NPT,), jnp.float32),
        pltpu.VMEM((NPT,), jnp.float32),
        pltpu.VMEM((NUM_GRAPHS,), jnp.float32),
        pltpu.VMEM((16, NUM_GRAPHS), jnp.float32),
        pltpu.SemaphoreType.DMA,
        pltpu.SemaphoreType.DMA,
        pltpu.VMEM_SHARED((16, NP), jnp.float32),
        pltpu.VMEM_SHARED((NP,), jnp.float32),
        pltpu.VMEM_SHARED((16, 128), jnp.float32),
    ],
)
def _sc_kernel(x_p, src, dst, batch_p, embed_flat, w_flat, b_bcast, out,
               *scratch):
    _sc_body(x_p, src, dst, batch_p, embed_flat, w_flat, b_bcast, out,
             *scratch)


def kernel(x, edge_index, batch, embed_table, W, b):
    x_p = jnp.pad(x[:, 0], (0, NP - N_NODES))
    batch_p = jnp.pad(batch, (0, NP - N_NODES))
    return _sc_kernel(x_p, edge_index[0], edge_index[1], batch_p,
                      embed_table.reshape(-1), W.reshape(-1),
                      jnp.broadcast_to(b, (16,)))


# speedup vs baseline: 171.7668x; 1.3351x over previous
import functools

import jax
import jax.numpy as jnp
from jax import lax
from jax.experimental import pallas as pl
from jax.experimental.pallas import tpu as pltpu
from jax.experimental.pallas import tpu_sc as plsc

N_NODES = 10000
N_EDGES = 320000
HIDDEN = 128
NUM_GRAPHS = 64
VOCAB = 28

NP = 10240
NPT = NP // 16
EPT = N_EDGES // 16


def _vrsqrt(d):
    i = lax.bitcast_convert_type(d, jnp.int32)
    i = jnp.int32(0x5F3759DF) - lax.shift_right_arithmetic(i, 1)
    y = lax.bitcast_convert_type(i, jnp.float32)
    for _ in range(3):
        y = y * (1.5 - 0.5 * d * y * y)
    return y


def _sc_body(x_hbm, ei_hbm, bt_hbm, emb_hbm, w_hbm, b_hbm,
             out_hbm,
             embed_v, w_v, b_v, zvec_v, s_v, hist_v, acc_v, src_v, dst_v,
             xv_v, btv_v, mat_v, dinv_v, z_v, out64_v, outm_v,
             sem_src, sem_dst,
             sh_part, sh_s, sh_out):
    sid = lax.axis_index("s")
    viota = lax.iota(jnp.int32, 16)
    zf = jnp.zeros((16,), jnp.float32)
    ones = jnp.ones((16,), jnp.float32)
    nbase = sid * NPT
    ebase = sid * EPT

    with jax.named_scope("ph_stage"):
        cp_dst = pltpu.async_copy(
            ei_hbm.at[pl.ds(N_EDGES + ebase, EPT)], dst_v, sem_dst)
        cp_src = pltpu.async_copy(
            ei_hbm.at[pl.ds(ebase, EPT)], src_v, sem_src)
        NTAIL = N_NODES - 15 * NPT
        @pl.when(sid < 15)
        def _():
            pltpu.sync_copy(x_hbm.at[pl.ds(nbase, NPT)], xv_v)
            pltpu.sync_copy(bt_hbm.at[pl.ds(nbase, NPT)], btv_v)
        @pl.when(sid == 15)
        def _():
            pltpu.sync_copy(x_hbm.at[pl.ds(15 * NPT, NTAIL)],
                            xv_v.at[pl.ds(0, NTAIL)])
            pltpu.sync_copy(bt_hbm.at[pl.ds(15 * NPT, NTAIL)],
                            btv_v.at[pl.ds(0, NTAIL)])
            zi = jnp.zeros((16,), jnp.int32)
            for q in range(NTAIL // 16, NPT // 16):
                xv_v[pl.ds(q * 16, 16)] = zi
                btv_v[pl.ds(q * 16, 16)] = zi
        pltpu.sync_copy(emb_hbm, embed_v.at[pl.ds(0, VOCAB * HIDDEN)])
        pltpu.sync_copy(w_hbm, w_v)
        pltpu.sync_copy(b_hbm, b_v.at[pl.ds(0, 1)])

    with jax.named_scope("ph_zvec"):
        idx0 = viota * HIDDEN
        idx1 = idx0 + 16 * HIDDEN

        def zstep(kk, carry):
            z0, z1 = carry
            wv = w_v[pl.ds(kk * 16, 16)]
            base = kk * 16
            for j in range(16):
                wk = wv[j]
                c0 = plsc.load_gather(embed_v, [idx0 + (base + j)])
                c1 = plsc.load_gather(embed_v, [idx1 + (base + j)])
                z0 = z0 + c0 * wk
                z1 = z1 + c1 * wk
            return (z0, z1)

        zv0, zv1 = lax.fori_loop(0, HIDDEN // 16, zstep, (zf, zf))
        zvec_v[pl.ds(0, 16)] = zv0
        zvec_v[pl.ds(16, 16)] = zv1

    with jax.named_scope("ph_zero"):
        @plsc.parallel_loop(0, NP // 16, unroll=8)
        def _(i):
            hist_v[pl.ds(i * 16, 16)] = zf
            acc_v[pl.ds(i * 16, 16)] = zf

    with jax.named_scope("ph_hist"):
        cp_dst.wait()

        @plsc.parallel_loop(0, EPT // 16, unroll=8)
        def _(i):
            dv = dst_v[pl.ds(i * 16, 16)]
            plsc.addupdate_scatter(hist_v, [dv], ones)

    with jax.named_scope("ph_degcomb"):
        pltpu.sync_copy(hist_v, sh_part.at[sid])
        plsc.subcore_barrier()
        pltpu.sync_copy(sh_part.at[:, pl.ds(nbase, NPT)], mat_v)

        @plsc.parallel_loop(0, NPT // 16, unroll=2)
        def _(i):
            sl = pl.ds(i * 16, 16)
            a = mat_v[0, sl]
            for t in range(1, 16):
                a = a + mat_v[t, sl]
            d = a + 1.0
            di = _vrsqrt(d)
            dinv_v[sl] = di
            zl = plsc.load_gather(zvec_v, [xv_v[sl]])
            z_v[sl] = zl
            s_v[pl.ds(nbase + i * 16, 16)] = di * zl

        pltpu.sync_copy(s_v.at[pl.ds(nbase, NPT)],
                        sh_s.at[pl.ds(nbase, NPT)])
        plsc.subcore_barrier()
        pltpu.sync_copy(sh_s, s_v)

    with jax.named_scope("ph_edge"):
        cp_src.wait()

        @plsc.parallel_loop(0, EPT // 16, unroll=8)
        def _(i):
            sl = pl.ds(i * 16, 16)
            m = plsc.load_gather(s_v, [src_v[sl]])
            plsc.addupdate_scatter(acc_v, [dst_v[sl]], m)

    with jax.named_scope("ph_final"):
        pltpu.sync_copy(acc_v, sh_part.at[sid])
        plsc.subcore_barrier()
        pltpu.sync_copy(sh_part.at[:, pl.ds(nbase, NPT)], mat_v)
        for q in range(NUM_GRAPHS // 16):
            out64_v[pl.ds(q * 16, 16)] = zf
        bvec = plsc.load_gather(b_v, [jnp.zeros((16,), jnp.int32)])

        @plsc.parallel_loop(0, NPT // 16, unroll=2)
        def _(i):
            sl = pl.ds(i * 16, 16)
            a = mat_v[0, sl]
            for t in range(1, 16):
                a = a + mat_v[t, sl]
            di = dinv_v[sl]
            yv = di * a + di * di * z_v[sl] + bvec
            gid = nbase + i * 16 + viota
            plsc.addupdate_scatter(out64_v, [btv_v[sl]], yv,
                                   mask=gid < N_NODES)

    with jax.named_scope("ph_out"):
        pltpu.sync_copy(out64_v, sh_out.at[sid, pl.ds(0, NUM_GRAPHS)])
        plsc.subcore_barrier()

        @pl.when(sid == 0)
        def _():
            for t in range(16):
                pltpu.sync_copy(sh_out.at[t, pl.ds(0, NUM_GRAPHS)],
                                outm_v.at[t])
            for q in range(NUM_GRAPHS // 16):
                a = outm_v[0, pl.ds(q * 16, 16)]
                for t in range(1, 16):
                    a = a + outm_v[t, pl.ds(q * 16, 16)]
                out64_v[pl.ds(q * 16, 16)] = a
            pltpu.sync_copy(out64_v, out_hbm)


@functools.partial(
    pl.kernel,
    out_type=jax.ShapeDtypeStruct((NUM_GRAPHS,), jnp.float32),
    mesh=plsc.VectorSubcoreMesh(core_axis_name="c", subcore_axis_name="s",
                                num_cores=1, num_subcores=16),
    compiler_params=pltpu.CompilerParams(needs_layout_passes=False),
    scratch_types=[
        pltpu.VMEM((4096,), jnp.float32),
        pltpu.VMEM((HIDDEN,), jnp.float32),
        pltpu.VMEM((16,), jnp.float32),
        pltpu.VMEM((32,), jnp.float32),
        pltpu.VMEM((NP,), jnp.float32),
        pltpu.VMEM((NP,), jnp.float32),
        pltpu.VMEM((NP,), jnp.float32),
        pltpu.VMEM((EPT,), jnp.int32),
        pltpu.VMEM((EPT,), jnp.int32),
        pltpu.VMEM((NPT,), jnp.int32),
        pltpu.VMEM((NPT,), jnp.int32),
        pltpu.VMEM((16, NPT), jnp.float32),
        pltpu.VMEM((NPT,), jnp.float32),
        pltpu.VMEM((NPT,), jnp.float32),
        pltpu.VMEM((NUM_GRAPHS,), jnp.float32),
        pltpu.VMEM((16, NUM_GRAPHS), jnp.float32),
        pltpu.SemaphoreType.DMA,
        pltpu.SemaphoreType.DMA,
        pltpu.VMEM_SHARED((16, NP), jnp.float32),
        pltpu.VMEM_SHARED((NP,), jnp.float32),
        pltpu.VMEM_SHARED((16, 128), jnp.float32),
    ],
)
def _sc_kernel(x_f, ei, batch, embed_flat, w_flat, b, out, *scratch):
    _sc_body(x_f, ei, batch, embed_flat, w_flat, b, out, *scratch)


def kernel(x, edge_index, batch, embed_table, W, b):
    return _sc_kernel(x.reshape(-1), edge_index.reshape(-1), batch,
                      embed_table.reshape(-1), W.reshape(-1), b)


# speedup vs baseline: 186.6466x; 1.0866x over previous
import functools

import jax
import jax.numpy as jnp
from jax import lax
from jax.experimental import pallas as pl
from jax.experimental.pallas import tpu as pltpu
from jax.experimental.pallas import tpu_sc as plsc

N_NODES = 10000
N_EDGES = 320000
HIDDEN = 128
NUM_GRAPHS = 64
VOCAB = 28

NP = 10240
NPT = NP // 16
EPB = 19968
EREM = N_EDGES - 16 * EPB


def _vrsqrt(d):
    i = lax.bitcast_convert_type(d, jnp.int32)
    i = jnp.int32(0x5F3759DF) - lax.shift_right_arithmetic(i, 1)
    y = lax.bitcast_convert_type(i, jnp.float32)
    for _ in range(3):
        y = y * (1.5 - 0.5 * d * y * y)
    return y


def _sc_body(x_hbm, ei_hbm, bt_hbm, emb_hbm, w_hbm, b_hbm,
             out_hbm,
             embed_v, w_v, b_v, zvec_v, s_v, hist_v, acc_v, ed_v, ed2_v,
             xv_v, btv_v, mat_v, dinv_v, z_v, out64_v, outm_v, zmat_v,
             sem_ed, sem_ed2, sem_x, sem_bt,
             sh_part, sh_s, sh_out):
    sid = lax.axis_index("s")
    viota = lax.iota(jnp.int32, 16)
    zf = jnp.zeros((16,), jnp.float32)
    ones = jnp.ones((16,), jnp.float32)
    nbase = sid * NPT
    ebase = sid * EPB

    with jax.named_scope("ph_stage"):
        cp_ed = pltpu.async_copy(ei_hbm.at[:, pl.ds(ebase, EPB)], ed_v,
                                 sem_ed)
        @pl.when(sid == 0)
        def _():
            cp2 = pltpu.async_copy(
                ei_hbm.at[:, pl.ds(16 * EPB, EREM)], ed2_v, sem_ed2)
            del cp2
        NTAIL = N_NODES - 15 * NPT
        @pl.when(sid < 15)
        def _():
            cp1 = pltpu.async_copy(x_hbm.at[pl.ds(nbase, NPT)], xv_v,
                                   sem_x)
            cp2 = pltpu.async_copy(bt_hbm.at[pl.ds(nbase, NPT)], btv_v,
                                   sem_bt)
            del cp1, cp2
        @pl.when(sid == 15)
        def _():
            cp1 = pltpu.async_copy(x_hbm.at[pl.ds(15 * NPT, NTAIL)],
                                   xv_v.at[pl.ds(0, NTAIL)], sem_x)
            cp2 = pltpu.async_copy(bt_hbm.at[pl.ds(15 * NPT, NTAIL)],
                                   btv_v.at[pl.ds(0, NTAIL)], sem_bt)
            del cp1, cp2
            zi = jnp.zeros((16,), jnp.int32)
            for q in range(NTAIL // 16, NPT // 16):
                xv_v[pl.ds(q * 16, 16)] = zi
                btv_v[pl.ds(q * 16, 16)] = zi
        pltpu.sync_copy(emb_hbm, embed_v.at[pl.ds(0, VOCAB * HIDDEN)])
        pltpu.sync_copy(w_hbm, w_v)
        pltpu.sync_copy(b_hbm, b_v.at[pl.ds(0, 1)])

    with jax.named_scope("ph_zvec"):
        idx0 = viota * HIDDEN
        idx1 = idx0 + 16 * HIDDEN
        kbase = sid * 8
        wv = w_v[pl.ds((sid // 2) * 16, 16)]

        def zpart(joff):
            z0, z1 = zf, zf
            for jj in range(8):
                wk = wv[joff + jj]
                k = kbase + jj
                c0 = plsc.load_gather(embed_v, [idx0 + k])
                c1 = plsc.load_gather(embed_v, [idx1 + k])
                z0 = z0 + c0 * wk
                z1 = z1 + c1 * wk
            zvec_v[pl.ds(0, 16)] = z0
            zvec_v[pl.ds(16, 16)] = z1

        @pl.when(sid % 2 == 0)
        def _():
            zpart(0)

        @pl.when(sid % 2 == 1)
        def _():
            zpart(8)

        pltpu.sync_copy(zvec_v, sh_out.at[sid, pl.ds(0, 32)])
        plsc.subcore_barrier()
        for t in range(16):
            pltpu.sync_copy(sh_out.at[t, pl.ds(0, 32)], zmat_v.at[t])
        for h in range(2):
            a = zmat_v[0, pl.ds(h * 16, 16)]
            for t in range(1, 16):
                a = a + zmat_v[t, pl.ds(h * 16, 16)]
            zvec_v[pl.ds(h * 16, 16)] = a

    with jax.named_scope("ph_zero"):
        @plsc.parallel_loop(0, NP // 16, unroll=8)
        def _(i):
            hist_v[pl.ds(i * 16, 16)] = zf
            acc_v[pl.ds(i * 16, 16)] = zf

    with jax.named_scope("ph_hist"):
        cp_ed.wait()

        @plsc.parallel_loop(0, EPB // 16, unroll=8)
        def _(i):
            dv = ed_v[1, pl.ds(i * 16, 16)]
            plsc.addupdate_scatter(hist_v, [dv], ones)

        @pl.when(sid == 0)
        def _():
            pltpu.make_async_copy(ei_hbm.at[:, pl.ds(16 * EPB, EREM)],
                                  ed2_v, sem_ed2).wait()

            @plsc.parallel_loop(0, EREM // 16, unroll=8)
            def _(i):
                dv = ed2_v[1, pl.ds(i * 16, 16)]
                plsc.addupdate_scatter(hist_v, [dv], ones)

    with jax.named_scope("ph_degcomb"):
        NTAIL = N_NODES - 15 * NPT
        @pl.when(sid < 15)
        def _():
            pltpu.make_async_copy(x_hbm.at[pl.ds(nbase, NPT)], xv_v,
                                  sem_x).wait()
        @pl.when(sid == 15)
        def _():
            pltpu.make_async_copy(x_hbm.at[pl.ds(15 * NPT, NTAIL)],
                                  xv_v.at[pl.ds(0, NTAIL)], sem_x).wait()
        pltpu.sync_copy(hist_v, sh_part.at[sid])
        plsc.subcore_barrier()
        pltpu.sync_copy(sh_part.at[:, pl.ds(nbase, NPT)], mat_v)

        @plsc.parallel_loop(0, NPT // 16, unroll=2)
        def _(i):
            sl = pl.ds(i * 16, 16)
            a = mat_v[0, sl]
            for t in range(1, 16):
                a = a + mat_v[t, sl]
            d = a + 1.0
            di = _vrsqrt(d)
            dinv_v[sl] = di
            zl = plsc.load_gather(zvec_v, [xv_v[sl]])
            z_v[sl] = zl
            s_v[pl.ds(nbase + i * 16, 16)] = di * zl

        pltpu.sync_copy(s_v.at[pl.ds(nbase, NPT)],
                        sh_s.at[pl.ds(nbase, NPT)])
        plsc.subcore_barrier()
        pltpu.sync_copy(sh_s, s_v)

    with jax.named_scope("ph_edge"):
        @plsc.parallel_loop(0, EPB // 16, unroll=8)
        def _(i):
            sl = pl.ds(i * 16, 16)
            m = plsc.load_gather(s_v, [ed_v[0, sl]])
            plsc.addupdate_scatter(acc_v, [ed_v[1, sl]], m)

        @pl.when(sid == 0)
        def _():
            @plsc.parallel_loop(0, EREM // 16, unroll=8)
            def _(i):
                sl = pl.ds(i * 16, 16)
                m = plsc.load_gather(s_v, [ed2_v[0, sl]])
                plsc.addupdate_scatter(acc_v, [ed2_v[1, sl]], m)

    with jax.named_scope("ph_final"):
        @pl.when(sid < 15)
        def _():
            pltpu.make_async_copy(bt_hbm.at[pl.ds(nbase, NPT)], btv_v,
                                  sem_bt).wait()
        @pl.when(sid == 15)
        def _():
            pltpu.make_async_copy(
                bt_hbm.at[pl.ds(15 * NPT, N_NODES - 15 * NPT)],
                btv_v.at[pl.ds(0, N_NODES - 15 * NPT)], sem_bt).wait()
        pltpu.sync_copy(acc_v, sh_part.at[sid])
        plsc.subcore_barrier()
        pltpu.sync_copy(sh_part.at[:, pl.ds(nbase, NPT)], mat_v)
        for q in range(NUM_GRAPHS // 16):
            out64_v[pl.ds(q * 16, 16)] = zf
        bvec = plsc.load_gather(b_v, [jnp.zeros((16,), jnp.int32)])

        @plsc.parallel_loop(0, NPT // 16, unroll=2)
        def _(i):
            sl = pl.ds(i * 16, 16)
            a = mat_v[0, sl]
            for t in range(1, 16):
                a = a + mat_v[t, sl]
            di = dinv_v[sl]
            yv = di * a + di * di * z_v[sl] + bvec
            gid = nbase + i * 16 + viota
            plsc.addupdate_scatter(out64_v, [btv_v[sl]], yv,
                                   mask=gid < N_NODES)

    with jax.named_scope("ph_out"):
        pltpu.sync_copy(out64_v, sh_out.at[sid, pl.ds(0, NUM_GRAPHS)])
        plsc.subcore_barrier()

        @pl.when(sid == 0)
        def _():
            for t in range(16):
                pltpu.sync_copy(sh_out.at[t, pl.ds(0, NUM_GRAPHS)],
                                outm_v.at[t])
            for q in range(NUM_GRAPHS // 16):
                a = outm_v[0, pl.ds(q * 16, 16)]
                for t in range(1, 16):
                    a = a + outm_v[t, pl.ds(q * 16, 16)]
                out64_v[pl.ds(q * 16, 16)] = a
            pltpu.sync_copy(out64_v, out_hbm)


@functools.partial(
    pl.kernel,
    out_type=jax.ShapeDtypeStruct((NUM_GRAPHS,), jnp.float32),
    mesh=plsc.VectorSubcoreMesh(core_axis_name="c", subcore_axis_name="s",
                                num_cores=1, num_subcores=16),
    compiler_params=pltpu.CompilerParams(needs_layout_passes=False),
    scratch_types=[
        pltpu.VMEM((4096,), jnp.float32),
        pltpu.VMEM((HIDDEN,), jnp.float32),
        pltpu.VMEM((16,), jnp.float32),
        pltpu.VMEM((32,), jnp.float32),
        pltpu.VMEM((NP,), jnp.float32),
        pltpu.VMEM((NP,), jnp.float32),
        pltpu.VMEM((NP,), jnp.float32),
        pltpu.VMEM((2, EPB), jnp.int32),
        pltpu.VMEM((2, EREM), jnp.int32),
        pltpu.VMEM((NPT,), jnp.int32),
        pltpu.VMEM((NPT,), jnp.int32),
        pltpu.VMEM((16, NPT), jnp.float32),
        pltpu.VMEM((NPT,), jnp.float32),
        pltpu.VMEM((NPT,), jnp.float32),
        pltpu.VMEM((NUM_GRAPHS,), jnp.float32),
        pltpu.VMEM((16, NUM_GRAPHS), jnp.float32),
        pltpu.VMEM((16, 32), jnp.float32),
        pltpu.SemaphoreType.DMA,
        pltpu.SemaphoreType.DMA,
        pltpu.SemaphoreType.DMA,
        pltpu.SemaphoreType.DMA,
        pltpu.VMEM_SHARED((16, NP), jnp.float32),
        pltpu.VMEM_SHARED((NP,), jnp.float32),
        pltpu.VMEM_SHARED((16, 128), jnp.float32),
    ],
)
def _sc_kernel(x_f, ei, batch, embed_flat, w_flat, b, out, *scratch):
    _sc_body(x_f, ei, batch, embed_flat, w_flat, b, out, *scratch)


def kernel(x, edge_index, batch, embed_table, W, b):
    return _sc_kernel(x.reshape(-1), edge_index, batch,
                      embed_table.reshape(-1), W.reshape(-1), b)
